# baseline (speedup 1.0000x reference)
import functools

import jax
import jax.numpy as jnp
from jax import lax
from jax.experimental import pallas as pl
from jax.experimental.pallas import tpu as pltpu
from jax.experimental.pallas import tpu_sc as plsc

_R = 16
_NB = 2


def kernel(inputs, embeddings):
    batch, seq_len, dim = inputs.shape
    nw = 32
    spw = seq_len // nw
    ngrp = spw // _R
    nch = ngrp * batch
    cwords = _R * dim
    x1 = inputs.reshape(batch * seq_len * dim)
    e1 = embeddings.reshape(seq_len * dim)

    mesh = plsc.VectorSubcoreMesh(core_axis_name="c", subcore_axis_name="s")

    @functools.partial(
        pl.kernel,
        out_type=jax.ShapeDtypeStruct((batch * seq_len * dim,), jnp.float32),
        mesh=mesh,
        scratch_types=[
            pltpu.VMEM((_NB, cwords), jnp.float32),
            pltpu.VMEM((_NB, cwords), jnp.float32),
            pltpu.SemaphoreType.DMA((_NB,)),
            pltpu.SemaphoreType.DMA((_NB,)),
            pltpu.SemaphoreType.DMA((_NB,)),
        ],
    )
    def sc_add(x_hbm, emb_hbm, out_hbm, xb, eb, sin, semb, sout):
        wid = lax.axis_index("s") * 2 + lax.axis_index("c")
        s0 = wid * spw

        def word0(g):
            grp, b = divmod(g, batch)
            return (b * seq_len + s0 + grp * _R) * dim

        def start_in(g):
            return pltpu.async_copy(
                x_hbm.at[pl.ds(word0(g), cwords)], xb.at[g % _NB],
                sin.at[g % _NB])

        def start_out(g):
            return pltpu.async_copy(
                xb.at[g % _NB], out_hbm.at[pl.ds(word0(g), cwords)],
                sout.at[g % _NB])

        def start_emb(grp):
            return pltpu.async_copy(
                emb_hbm.at[pl.ds((s0 + grp * _R) * dim, cwords)],
                eb.at[grp % _NB], semb.at[grp % _NB])

        def add_chunk(g):
            xr, er = xb.at[g % _NB], eb.at[(g // batch) % _NB]

            def body(i, _):
                o = i * 16
                xr[pl.ds(o, 16)] = xr[pl.ds(o, 16)] + er[pl.ds(o, 16)]
                return 0

            lax.fori_loop(0, cwords // 16, body, 0, unroll=8)

        emb_d = {0: start_emb(0)}
        in_d, out_d = {0: start_in(0)}, {}
        for g in range(nch):
            grp, b = divmod(g, batch)
            in_d.pop(g).wait()
            if b == 0:
                emb_d.pop(grp).wait()
            nxt = g + 1
            if nxt < nch:
                if nxt % batch == 0 and (nxt // batch) < ngrp:
                    emb_d[nxt // batch] = start_emb(nxt // batch)
                if g - 1 in out_d:
                    out_d.pop(g - 1).wait()
                in_d[nxt] = start_in(nxt)
            add_chunk(g)
            out_d[g] = start_out(g)
        out_d.pop(nch - 1).wait()

    out = sc_add(x1, e1)
    return out.reshape(inputs.shape)


# speedup vs baseline: 7.2930x; 7.2930x over previous
import jax
import jax.numpy as jnp
from jax.experimental import pallas as pl


_S_BLK = 2048


def _add_kernel(x_ref, e_ref, o_ref):
    o_ref[...] = x_ref[...] + e_ref[...]


def kernel(inputs, embeddings):
    batch, seq_len, dim = inputs.shape
    pos = embeddings[:seq_len]
    ns = seq_len // _S_BLK
    return pl.pallas_call(
        _add_kernel,
        grid=(ns, batch),
        in_specs=[
            pl.BlockSpec((1, _S_BLK, dim), lambda s, b: (b, s, 0)),
            pl.BlockSpec((_S_BLK, dim), lambda s, b: (s, 0)),
        ],
        out_specs=pl.BlockSpec((1, _S_BLK, dim), lambda s, b: (b, s, 0)),
        out_shape=jax.ShapeDtypeStruct(inputs.shape, inputs.dtype),
    )(inputs, pos)
